# Initial kernel scaffold; baseline (speedup 1.0000x reference)
import functools

import jax
import jax.numpy as jnp
from jax import lax
from jax.experimental import pallas as pl
from jax.experimental.pallas import tpu as pltpu
from jax.experimental.pallas import tpu_sc as plsc

B = 4096
T = 50
D = 768
N = 1024
V = 32000

NC = 2
NS = 16
NW = NC * NS
BPW = B // NW
LANES = 16
DCH = D // LANES


def _sc_pool_kernel(toks_hbm, emb_hbm, z_hbm, idx_v, rows0, rows1,
                    acc0, acc1, g0, g1, o0, o1):
    wid = lax.axis_index("s") * NC + lax.axis_index("c")
    base = wid * BPW

    pltpu.sync_copy(toks_hbm.at[pl.ds(base, BPW)], idx_v)

    rows = (rows0, rows1)
    accs = (acc0, acc1)
    gsems = (g0, g1)
    osems = (o0, o1)

    pltpu.async_copy(emb_hbm.at[idx_v.at[0]], rows0, g0)

    def step(i, _):
        for bslot in (0, 1):
            e = 2 * i + bslot
            rbuf = rows[bslot]
            abuf = accs[bslot]

            pltpu.make_async_copy(emb_hbm.at[idx_v.at[e]], rbuf,
                                  gsems[bslot]).wait()

            nbuf = rows[1 - bslot]

            @pl.when(e + 1 < BPW)
            def _():
                pltpu.async_copy(emb_hbm.at[idx_v.at[e + 1]], nbuf,
                                 gsems[1 - bslot])

            @pl.when(e >= 2)
            def _():
                pltpu.make_async_copy(abuf, z_hbm.at[base + e - 2],
                                      osems[bslot]).wait()

            for k in range(DCH):
                abuf[pl.ds(k * LANES, LANES)] = rbuf[0, pl.ds(k * LANES, LANES)]

            def add_row(j, _):
                for k in range(DCH):
                    plsc.addupdate(abuf.at[pl.ds(k * LANES, LANES)],
                                   rbuf[j, pl.ds(k * LANES, LANES)])
                return ()

            lax.fori_loop(1, T, add_row, ())

            pltpu.async_copy(abuf, z_hbm.at[base + e], osems[bslot])
        return ()

    lax.fori_loop(0, BPW // 2, step, ())

    pltpu.make_async_copy(acc0, z_hbm.at[base + BPW - 2], o0).wait()
    pltpu.make_async_copy(acc1, z_hbm.at[base + BPW - 1], o1).wait()


def _sc_pool(toks, emb):
    mesh = plsc.VectorSubcoreMesh(core_axis_name="c", subcore_axis_name="s")
    return pl.kernel(
        _sc_pool_kernel,
        mesh=mesh,
        out_type=jax.ShapeDtypeStruct((B, D), jnp.float32),
        scratch_types=[
            pltpu.VMEM((BPW, T), jnp.int32),
            pltpu.VMEM((T, D), jnp.float32),
            pltpu.VMEM((T, D), jnp.float32),
            pltpu.VMEM((D,), jnp.float32),
            pltpu.VMEM((D,), jnp.float32),
            pltpu.SemaphoreType.DMA,
            pltpu.SemaphoreType.DMA,
            pltpu.SemaphoreType.DMA,
            pltpu.SemaphoreType.DMA,
        ],
    )(toks, emb)


BM = 512


def _proj_kernel(z_ref, w_ref, b_ref, o_ref):
    y = jnp.dot(z_ref[...], w_ref[...], preferred_element_type=jnp.float32)
    y = y * (1.0 / T) + b_ref[...]
    n = jnp.sqrt(jnp.sum(y * y, axis=1, keepdims=True))
    o_ref[...] = y / jnp.maximum(n, 1e-12)


def _tc_proj(z, W, b):
    return pl.pallas_call(
        _proj_kernel,
        grid=(B // BM,),
        in_specs=[
            pl.BlockSpec((BM, D), lambda i: (i, 0)),
            pl.BlockSpec((D, N), lambda i: (0, 0)),
            pl.BlockSpec((1, N), lambda i: (0, 0)),
        ],
        out_specs=pl.BlockSpec((BM, N), lambda i: (i, 0)),
        out_shape=jax.ShapeDtypeStruct((B, N), jnp.float32),
    )(z, W, b.reshape(1, N))


def kernel(toks, emb, W, b):
    toks = toks.astype(jnp.int32)
    z = _sc_pool(toks, emb)
    return _tc_proj(z, W, b)


# speedup vs baseline: 1.3421x; 1.3421x over previous
import functools

import jax
import jax.numpy as jnp
from jax import lax
from jax.experimental import pallas as pl
from jax.experimental.pallas import tpu as pltpu
from jax.experimental.pallas import tpu_sc as plsc

B = 4096
T = 50
TP = 56
D = 768
N = 1024
V = 32000

NC = 2
NS = 16
NW = NC * NS
BPW = B // NW
LANES = 16
DCH = D // LANES


def _sc_pool_kernel(toks_hbm, emb_hbm, z_hbm, idx_v, rows0, rows1,
                    acc0, acc1, g0, g1, o0, o1):
    wid = lax.axis_index("s") * NC + lax.axis_index("c")
    base = wid * BPW

    pltpu.sync_copy(toks_hbm.at[pl.ds(base, BPW)], idx_v)

    def idx_row(e):
        return idx_v.at[e, pl.ds(0, T)]

    rows = (rows0, rows1)
    accs = (acc0, acc1)
    gsems = (g0, g1)
    osems = (o0, o1)

    pltpu.async_copy(emb_hbm.at[idx_row(0)], rows0, g0)

    def step(i, _):
        for bslot in (0, 1):
            e = 2 * i + bslot
            rbuf = rows[bslot]
            abuf = accs[bslot]

            pltpu.make_async_copy(emb_hbm.at[idx_row(e)], rbuf,
                                  gsems[bslot]).wait()

            nbuf = rows[1 - bslot]

            @pl.when(e + 1 < BPW)
            def _():
                pltpu.async_copy(emb_hbm.at[idx_row(e + 1)], nbuf,
                                 gsems[1 - bslot])

            @pl.when(e >= 2)
            def _():
                pltpu.make_async_copy(abuf, z_hbm.at[base + e - 2],
                                      osems[bslot]).wait()

            for k in range(DCH):
                abuf[pl.ds(k * LANES, LANES)] = rbuf[0, pl.ds(k * LANES, LANES)]

            def add_row(j, _):
                for k in range(DCH):
                    plsc.addupdate(abuf.at[pl.ds(k * LANES, LANES)],
                                   rbuf[j, pl.ds(k * LANES, LANES)])
                return ()

            lax.fori_loop(1, T, add_row, ())

            pltpu.async_copy(abuf, z_hbm.at[base + e], osems[bslot])
        return ()

    lax.fori_loop(0, BPW // 2, step, ())

    pltpu.make_async_copy(acc0, z_hbm.at[base + BPW - 2], o0).wait()
    pltpu.make_async_copy(acc1, z_hbm.at[base + BPW - 1], o1).wait()


def _sc_pool(toks, emb):
    mesh = plsc.VectorSubcoreMesh(core_axis_name="c", subcore_axis_name="s")
    return pl.kernel(
        _sc_pool_kernel,
        mesh=mesh,
        out_type=jax.ShapeDtypeStruct((B, D), jnp.float32),
        scratch_types=[
            pltpu.VMEM((BPW, TP), jnp.int32),
            pltpu.VMEM((T, D), jnp.float32),
            pltpu.VMEM((T, D), jnp.float32),
            pltpu.VMEM((D,), jnp.float32),
            pltpu.VMEM((D,), jnp.float32),
            pltpu.SemaphoreType.DMA,
            pltpu.SemaphoreType.DMA,
            pltpu.SemaphoreType.DMA,
            pltpu.SemaphoreType.DMA,
        ],
    )(toks, emb)


BM = 512


def _proj_kernel(z_ref, w_ref, b_ref, o_ref):
    y = jnp.dot(z_ref[...], w_ref[...], preferred_element_type=jnp.float32)
    y = y * (1.0 / T) + b_ref[...]
    n = jnp.sqrt(jnp.sum(y * y, axis=1, keepdims=True))
    o_ref[...] = y / jnp.maximum(n, 1e-12)


def _tc_proj(z, W, b):
    return pl.pallas_call(
        _proj_kernel,
        grid=(B // BM,),
        in_specs=[
            pl.BlockSpec((BM, D), lambda i: (i, 0)),
            pl.BlockSpec((D, N), lambda i: (0, 0)),
            pl.BlockSpec((1, N), lambda i: (0, 0)),
        ],
        out_specs=pl.BlockSpec((BM, N), lambda i: (i, 0)),
        out_shape=jax.ShapeDtypeStruct((B, N), jnp.float32),
    )(z, W, b.reshape(1, N))


def kernel(toks, emb, W, b):
    toks = jnp.pad(toks.astype(jnp.int32), ((0, 0), (0, TP - T)))
    z = _sc_pool(toks, emb)
    return _tc_proj(z, W, b)


# speedup vs baseline: 4.3555x; 3.2453x over previous
import functools

import jax
import jax.numpy as jnp
from jax import lax
from jax.experimental import pallas as pl
from jax.experimental.pallas import tpu as pltpu
from jax.experimental.pallas import tpu_sc as plsc

B = 4096
T = 50
TP = 56
D = 768
N = 1024
V = 32000

NC = 2
NS = 16
NW = NC * NS
BPW = B // NW
LANES = 16
DCH = D // LANES


def _sc_pool_kernel(toks_hbm, emb_hbm, z_hbm, idx_v, rows0, rows1,
                    acc0, acc1, g0, g1, o0, o1):
    wid = lax.axis_index("s") * NC + lax.axis_index("c")
    base = wid * BPW

    pltpu.sync_copy(toks_hbm.at[pl.ds(base, BPW)], idx_v)

    def idx_row(e):
        return idx_v.at[e, pl.ds(0, T)]

    rows = (rows0, rows1)
    accs = (acc0, acc1)
    gsems = (g0, g1)
    osems = (o0, o1)

    pltpu.async_copy(emb_hbm.at[idx_row(0)], rows0, g0)

    def step(i, _):
        for bslot in (0, 1):
            e = 2 * i + bslot
            rbuf = rows[bslot]
            abuf = accs[bslot]

            pltpu.make_async_copy(emb_hbm.at[idx_row(e)], rbuf,
                                  gsems[bslot]).wait()

            nbuf = rows[1 - bslot]

            @pl.when(e + 1 < BPW)
            def _():
                pltpu.async_copy(emb_hbm.at[idx_row(e + 1)], nbuf,
                                 gsems[1 - bslot])

            @pl.when(e >= 2)
            def _():
                pltpu.make_async_copy(abuf, z_hbm.at[base + e - 2],
                                      osems[bslot]).wait()

            KH = DCH // 2
            for half in range(2):
                koff = half * KH * LANES

                def add_row(j, accs, koff=koff):
                    return tuple(
                        accs[k] + rbuf[j, pl.ds(koff + k * LANES, LANES)]
                        for k in range(KH))

                init = tuple(rbuf[0, pl.ds(koff + k * LANES, LANES)]
                             for k in range(KH))
                sums = lax.fori_loop(1, T, add_row, init)
                for k in range(KH):
                    abuf[pl.ds(koff + k * LANES, LANES)] = sums[k]

            pltpu.async_copy(abuf, z_hbm.at[base + e], osems[bslot])
        return ()

    lax.fori_loop(0, BPW // 2, step, ())

    pltpu.make_async_copy(acc0, z_hbm.at[base + BPW - 2], o0).wait()
    pltpu.make_async_copy(acc1, z_hbm.at[base + BPW - 1], o1).wait()


def _sc_pool(toks, emb):
    mesh = plsc.VectorSubcoreMesh(core_axis_name="c", subcore_axis_name="s")
    return pl.kernel(
        _sc_pool_kernel,
        mesh=mesh,
        out_type=jax.ShapeDtypeStruct((B, D), jnp.float32),
        scratch_types=[
            pltpu.VMEM((BPW, TP), jnp.int32),
            pltpu.VMEM((T, D), jnp.float32),
            pltpu.VMEM((T, D), jnp.float32),
            pltpu.VMEM((D,), jnp.float32),
            pltpu.VMEM((D,), jnp.float32),
            pltpu.SemaphoreType.DMA,
            pltpu.SemaphoreType.DMA,
            pltpu.SemaphoreType.DMA,
            pltpu.SemaphoreType.DMA,
        ],
    )(toks, emb)


BM = 512


def _proj_kernel(z_ref, w_ref, b_ref, o_ref):
    y = jnp.dot(z_ref[...], w_ref[...], preferred_element_type=jnp.float32)
    y = y * (1.0 / T) + b_ref[...]
    n = jnp.sqrt(jnp.sum(y * y, axis=1, keepdims=True))
    o_ref[...] = y / jnp.maximum(n, 1e-12)


def _tc_proj(z, W, b):
    return pl.pallas_call(
        _proj_kernel,
        grid=(B // BM,),
        in_specs=[
            pl.BlockSpec((BM, D), lambda i: (i, 0)),
            pl.BlockSpec((D, N), lambda i: (0, 0)),
            pl.BlockSpec((1, N), lambda i: (0, 0)),
        ],
        out_specs=pl.BlockSpec((BM, N), lambda i: (i, 0)),
        out_shape=jax.ShapeDtypeStruct((B, N), jnp.float32),
    )(z, W, b.reshape(1, N))


def kernel(toks, emb, W, b):
    toks = jnp.pad(toks.astype(jnp.int32), ((0, 0), (0, TP - T)))
    z = _sc_pool(toks, emb)
    return _tc_proj(z, W, b)
